# baseline (speedup 1.0000x reference)
import functools

import jax
import jax.numpy as jnp
from jax import lax
from jax.experimental import pallas as pl
from jax.experimental.pallas import tpu as pltpu
from jax.experimental.pallas import tpu_sc as plsc

B = 16384
D = 32
L = 16
NUM_CORES = 2
NUM_SUBCORES = 16
NW = NUM_CORES * NUM_SUBCORES
BPW = B // NW
GROUPS = BPW // L

_mesh = plsc.VectorSubcoreMesh(core_axis_name="c", subcore_axis_name="s")


@functools.partial(
    pl.kernel,
    mesh=_mesh,
    out_type=jax.ShapeDtypeStruct((B,), jnp.float32),
    scratch_types=[
        pltpu.VMEM((BPW,), jnp.int32),
        pltpu.VMEM((BPW,), jnp.int32),
        pltpu.VMEM((BPW, D), jnp.float32),
        pltpu.VMEM((BPW, D), jnp.float32),
        pltpu.VMEM((BPW,), jnp.float32),
        pltpu.VMEM((BPW,), jnp.float32),
        pltpu.VMEM((BPW,), jnp.float32),
        pltpu.SemaphoreType.DMA,
    ],
    compiler_params=pltpu.CompilerParams(
        needs_layout_passes=False, use_tc_tiling_on_sc=False
    ),
)
def _mf_sc(user_hbm, mission_hbm, uemb_hbm, memb_hbm, ubias_hbm, mbias_hbm,
           out_hbm, uidx_v, midx_v, urows_v, mrows_v, ub_v, mb_v, o_v, sem):
    wid = lax.axis_index("s") * NUM_CORES + lax.axis_index("c")
    base = wid * BPW

    pltpu.sync_copy(user_hbm.at[pl.ds(base, BPW)], uidx_v)
    pltpu.sync_copy(mission_hbm.at[pl.ds(base, BPW)], midx_v)

    cp_u = pltpu.async_copy(uemb_hbm.at[uidx_v], urows_v, sem)
    cp_m = pltpu.async_copy(memb_hbm.at[midx_v], mrows_v, sem)
    cp_ub = pltpu.async_copy(ubias_hbm.at[uidx_v], ub_v, sem)
    cp_mb = pltpu.async_copy(mbias_hbm.at[midx_v], mb_v, sem)
    cp_u.wait()
    cp_m.wait()
    cp_ub.wait()
    cp_mb.wait()

    iota = lax.iota(jnp.int32, L)

    def group_body(g, carry):
        rows = g * L + iota
        acc = ub_v[pl.ds(g * L, L)] + mb_v[pl.ds(g * L, L)]
        for d in range(D):
            d_vec = jnp.full((L,), d, jnp.int32)
            u_col = plsc.load_gather(urows_v, [rows, d_vec])
            m_col = plsc.load_gather(mrows_v, [rows, d_vec])
            acc = acc + u_col * m_col
        o_v[pl.ds(g * L, L)] = acc
        return carry

    lax.fori_loop(0, GROUPS, group_body, 0)

    pltpu.sync_copy(o_v, out_hbm.at[pl.ds(base, BPW)])


def kernel(user, mission, user_embedding, mission_embedding, user_bias, mission_bias):
    ub = user_bias.reshape(-1)
    mb = mission_bias.reshape(-1)
    return _mf_sc(user, mission, user_embedding, mission_embedding, ub, mb)
